# baseline (speedup 1.0000x reference)
import functools

import jax
import jax.numpy as jnp
from jax import lax
from jax.experimental import pallas as pl
from jax.experimental.pallas import tpu as pltpu
from jax.experimental.pallas import tpu_sc as plsc

N_NODES = 10000
N_EDGES = 320000
D_FEAT = 128
D_EDGE = 16
D_GLOBAL = 128
D_OUT = D_EDGE + 2 * D_FEAT + D_GLOBAL
ROW_R = D_EDGE
ROW_S = D_EDGE + D_FEAT
ROW_G = D_EDGE + 2 * D_FEAT
D_BUF = ROW_G

NUM_CORES = 2
NUM_SUBCORES = 16
NW = NUM_CORES * NUM_SUBCORES
CHUNK = 128
N_CHUNKS = N_EDGES // CHUNK
L = 16

_mesh = plsc.VectorSubcoreMesh(core_axis_name="c", subcore_axis_name="s")


@functools.partial(
    pl.kernel,
    out_type=jax.ShapeDtypeStruct((D_OUT, N_EDGES), jnp.float32),
    mesh=_mesh,
    compiler_params=pltpu.CompilerParams(needs_layout_passes=False),
    scratch_types=[
        [pltpu.VMEM((CHUNK,), jnp.int32)] * 2,
        [pltpu.VMEM((CHUNK,), jnp.int32)] * 2,
        pltpu.VMEM((CHUNK, D_FEAT), jnp.float32),
        pltpu.VMEM((CHUNK, D_FEAT), jnp.float32),
        [pltpu.VMEM((D_BUF, CHUNK), jnp.float32)] * 2,
        pltpu.VMEM((D_GLOBAL, CHUNK), jnp.float32),
        pltpu.VMEM((D_GLOBAL,), jnp.float32),
        pltpu.SemaphoreType.DMA,
        pltpu.SemaphoreType.DMA,
        [pltpu.SemaphoreType.DMA] * 2,
        [pltpu.SemaphoreType.DMA] * 2,
    ],
)
def _edge_block_sc(
    edges_t_hbm, nodes_hbm, glob_hbm, recv_hbm, send_hbm, out_hbm,
    idx_r, idx_s, stage_r, stage_s, buf, glob_v, grow_v,
    isem, gsem, esem, wsem,
):
    wid = lax.axis_index("s") * NUM_CORES + lax.axis_index("c")
    nk = jnp.where(wid < N_CHUNKS % NW, N_CHUNKS // NW + 1, N_CHUNKS // NW)

    pltpu.sync_copy(glob_hbm, grow_v)

    def fill_glob(c, carry):
        v = plsc.load_gather(grow_v, [jnp.full((L,), c, jnp.int32)])
        for b in range(CHUNK // L):
            glob_v[c, pl.ds(b * L, L)] = v
        return carry

    lax.fori_loop(0, D_GLOBAL, fill_glob, 0)

    iota = lax.iota(jnp.int32, L)

    def idx_descs(sl, j):
        ebase = (wid + NW * j) * CHUNK
        return [
            pltpu.make_async_copy(recv_hbm.at[pl.ds(ebase, CHUNK)], idx_r[sl], isem),
            pltpu.make_async_copy(send_hbm.at[pl.ds(ebase, CHUNK)], idx_s[sl], isem),
        ]

    def gather_descs(sl):
        return [
            pltpu.make_async_copy(nodes_hbm.at[idx_r[sl]], stage_r, gsem),
            pltpu.make_async_copy(nodes_hbm.at[idx_s[sl]], stage_s, gsem),
        ]

    def edge_desc(sl, j):
        ebase = (wid + NW * j) * CHUNK
        return pltpu.make_async_copy(
            edges_t_hbm.at[:, pl.ds(ebase, CHUNK)], buf[sl].at[pl.ds(0, D_EDGE), :],
            esem[sl])

    def write_descs(sl, j):
        ebase = (wid + NW * j) * CHUNK
        cols = pl.ds(ebase, CHUNK)
        return [
            pltpu.make_async_copy(buf[sl], out_hbm.at[pl.ds(0, D_BUF), cols], wsem[sl]),
            pltpu.make_async_copy(glob_v, out_hbm.at[pl.ds(ROW_G, D_GLOBAL), cols], wsem[sl]),
        ]

    def transpose_into(sl):
        def trow(e, carry):
            col = jnp.full((L,), e, jnp.int32)
            for c0 in range(0, D_FEAT, L):
                vr = stage_r[e, pl.ds(c0, L)]
                plsc.store_scatter(buf[sl], [ROW_R + c0 + iota, col], vr)
                vs = stage_s[e, pl.ds(c0, L)]
                plsc.store_scatter(buf[sl], [ROW_S + c0 + iota, col], vs)
            return carry

        lax.fori_loop(0, CHUNK, trow, 0)

    def do_chunk(sl, j):
        osl = 1 - sl

        @pl.when(j >= 2)
        def _():
            for d in write_descs(sl, j):
                d.wait()

        @pl.when(j + 1 < nk)
        def _():
            for d in idx_descs(osl, j + 1):
                d.start()

        edge_desc(sl, j).start()
        for d in gather_descs(sl):
            d.wait()
        transpose_into(sl)

        @pl.when(j + 1 < nk)
        def _():
            for d in idx_descs(osl, j + 1):
                d.wait()
            for d in gather_descs(osl):
                d.start()

        edge_desc(sl, j).wait()
        for d in write_descs(sl, j):
            d.start()

    for d in idx_descs(0, jnp.int32(0)):
        d.start()
    for d in idx_descs(0, jnp.int32(0)):
        d.wait()
    for d in gather_descs(0):
        d.start()

    def body(j, carry):
        @pl.when(j % 2 == 0)
        def _():
            do_chunk(0, j)

        @pl.when(j % 2 == 1)
        def _():
            do_chunk(1, j)

        return carry

    lax.fori_loop(0, nk, body, 0)

    for sl in range(2):
        for d in write_descs(sl, jnp.int32(0)):
            d.wait()


def kernel(edges_data, nodes_data, global_data, receivers, senders):
    out_t = _edge_block_sc(
        edges_data.T,
        nodes_data,
        global_data,
        receivers.astype(jnp.int32),
        senders.astype(jnp.int32),
    )
    return out_t.T
